# baseline (speedup 1.0000x reference)
import functools
import jax
import jax.numpy as jnp
from jax import lax
from jax.experimental import pallas as pl
from jax.experimental.pallas import tpu as pltpu
from jax.experimental.pallas import tpu_sc as plsc

B = 4096
PEP_LEN = 21
MHC_LEN = 34
CORE_LEN = 15
EMB = 128
PAD = 3
VOCAB = 30

NC = 2
NS = 16
NW = NC * NS

ROWS_W = B // NW
MSK_W = ROWS_W * CORE_LEN

GP = 4
NBUF = 4
PEP_ITERS = ROWS_W // (GP * NBUF)

_mesh = plsc.VectorSubcoreMesh(core_axis_name="c", subcore_axis_name="s")


@functools.partial(
    pl.kernel,
    mesh=_mesh,
    out_type=[
        jax.ShapeDtypeStruct((B, PEP_LEN, EMB), jnp.float32),
        jax.ShapeDtypeStruct((B * CORE_LEN,), jnp.int32),
    ],
    scratch_types=[
        pltpu.VMEM((ROWS_W, PEP_LEN), jnp.int32),
        pltpu.VMEM((MSK_W + 16,), jnp.int32),
        pltpu.VMEM_SHARED((VOCAB, EMB), jnp.float32),
    ]
    + [pltpu.VMEM((GP, PEP_LEN, EMB), jnp.float32) for _ in range(NBUF)]
    + [pltpu.SemaphoreType.DMA for _ in range(2 * NBUF + 1)],
)
def _pep_lookup(pep_x2, pep_tab, pep_out, msk_out,
                pep_idx2_v, msk_v, pep_tab_v, *bufs_and_sems):
    bufs = list(bufs_and_sems[:NBUF])
    gsems = list(bufs_and_sems[NBUF:2 * NBUF])
    osems = list(bufs_and_sems[2 * NBUF:3 * NBUF])
    msem = bufs_and_sems[3 * NBUF]

    wid = lax.axis_index("s") * NC + lax.axis_index("c")
    row0 = wid * ROWS_W

    pltpu.sync_copy(pep_x2.at[pl.ds(row0, ROWS_W)], pep_idx2_v)

    @pl.when(lax.axis_index("s") == 0)
    def _stage_table():
        pltpu.sync_copy(pep_tab, pep_tab_v)

    plsc.subcore_barrier()

    def mask_body(b, carry):
        mv = pep_idx2_v[b, pl.ds(PAD, 16)]
        m = jnp.where(mv != jnp.zeros((16,), jnp.int32),
                      jnp.ones((16,), jnp.int32),
                      jnp.zeros((16,), jnp.int32))
        msk_v[pl.ds(b * CORE_LEN, 16)] = m
        return carry

    lax.fori_loop(0, ROWS_W, mask_body, 0)
    mcopy = pltpu.async_copy(msk_v.at[pl.ds(0, MSK_W)],
                             msk_out.at[pl.ds(wid * MSK_W, MSK_W)], msem)

    def body(gg, carry):
        g0 = gg * NBUF
        gcs = []
        for s in range(NBUF):
            b0 = (g0 + s) * GP
            cps = []
            for r in range(GP):
                cps.append(pltpu.async_copy(
                    pep_tab_v.at[pep_idx2_v.at[b0 + r]], bufs[s].at[r],
                    gsems[s]))
            gcs.append(cps)
        ocs = []
        for s in range(NBUF):
            for cp in gcs[s]:
                cp.wait()
            b0 = (g0 + s) * GP
            ocs.append(pltpu.async_copy(
                bufs[s], pep_out.at[pl.ds(row0 + b0, GP)], osems[s]))
        for oc in ocs:
            oc.wait()
        return carry

    lax.fori_loop(0, PEP_ITERS, body, 0)
    mcopy.wait()


GM = 2
MHC_ITERS = ROWS_W // (GM * NBUF)


@functools.partial(
    pl.kernel,
    mesh=_mesh,
    out_type=jax.ShapeDtypeStruct((B, MHC_LEN, EMB), jnp.float32),
    scratch_types=[
        pltpu.VMEM((ROWS_W, MHC_LEN), jnp.int32),
        pltpu.VMEM_SHARED((VOCAB, EMB), jnp.float32),
    ]
    + [pltpu.VMEM((GM, MHC_LEN, EMB), jnp.float32) for _ in range(NBUF)]
    + [pltpu.SemaphoreType.DMA for _ in range(2 * NBUF)],
)
def _mhc_lookup(mhc_x2, mhc_tab, mhc_out,
                mhc_idx2_v, mhc_tab_v, *bufs_and_sems):
    bufs = list(bufs_and_sems[:NBUF])
    gsems = list(bufs_and_sems[NBUF:2 * NBUF])
    osems = list(bufs_and_sems[2 * NBUF:3 * NBUF])

    wid = lax.axis_index("s") * NC + lax.axis_index("c")
    row0 = wid * ROWS_W

    pltpu.sync_copy(mhc_x2.at[pl.ds(row0, ROWS_W)], mhc_idx2_v)

    @pl.when(lax.axis_index("s") == 0)
    def _stage_table():
        pltpu.sync_copy(mhc_tab, mhc_tab_v)

    plsc.subcore_barrier()

    def body(gg, carry):
        g0 = gg * NBUF
        gcs = []
        for s in range(NBUF):
            b0 = (g0 + s) * GM
            cps = []
            for r in range(GM):
                cps.append(pltpu.async_copy(
                    mhc_tab_v.at[mhc_idx2_v.at[b0 + r]], bufs[s].at[r],
                    gsems[s]))
            gcs.append(cps)
        ocs = []
        for s in range(NBUF):
            for cp in gcs[s]:
                cp.wait()
            b0 = (g0 + s) * GM
            ocs.append(pltpu.async_copy(
                bufs[s], mhc_out.at[pl.ds(row0 + b0, GM)], osems[s]))
        for oc in ocs:
            oc.wait()
        return carry

    lax.fori_loop(0, MHC_ITERS, body, 0)


def kernel(peptide_x, peptide_esm_x, mhc_x, peptide_emb, mhc_emb):
    del peptide_esm_x
    mhc_out = _mhc_lookup(mhc_x.astype(jnp.int32), mhc_emb)
    pep_tab = peptide_emb + mhc_out[0, 0, 0] * 0.0
    pep_out, msk = _pep_lookup(peptide_x.astype(jnp.int32), pep_tab)
    masks = msk.reshape(B, CORE_LEN).astype(jnp.bool_)
    return (pep_out, masks, mhc_out)


# speedup vs baseline: 1.0467x; 1.0467x over previous
import functools
import jax
import jax.numpy as jnp
from jax import lax
from jax.experimental import pallas as pl
from jax.experimental.pallas import tpu as pltpu
from jax.experimental.pallas import tpu_sc as plsc

B = 4096
PEP_LEN = 21
MHC_LEN = 34
CORE_LEN = 15
EMB = 128
PAD = 3
VOCAB = 30

NC = 2
NS = 16
NW = NC * NS

ROWS_W = B // NW
MSK_W = ROWS_W * CORE_LEN

GP = 4
NBUF = 4
PEP_ITERS = ROWS_W // (GP * NBUF)

_mesh = plsc.VectorSubcoreMesh(core_axis_name="c", subcore_axis_name="s")


@functools.partial(
    pl.kernel,
    mesh=_mesh,
    out_type=[
        jax.ShapeDtypeStruct((B, PEP_LEN, EMB), jnp.float32),
        jax.ShapeDtypeStruct((B * CORE_LEN,), jnp.int32),
    ],
    scratch_types=[
        pltpu.VMEM((ROWS_W, PEP_LEN), jnp.int32),
        pltpu.VMEM((MSK_W + 16,), jnp.int32),
        pltpu.VMEM_SHARED((VOCAB, EMB), jnp.float32),
    ]
    + [pltpu.VMEM((GP, PEP_LEN, EMB), jnp.float32) for _ in range(NBUF)]
    + [pltpu.SemaphoreType.DMA for _ in range(2 * NBUF + 1)],
)
def _pep_lookup(pep_x2, pep_tab, pep_out, msk_out,
                pep_idx2_v, msk_v, pep_tab_v, *bufs_and_sems):
    bufs = list(bufs_and_sems[:NBUF])
    gsems = list(bufs_and_sems[NBUF:2 * NBUF])
    osems = list(bufs_and_sems[2 * NBUF:3 * NBUF])
    msem = bufs_and_sems[3 * NBUF]

    wid = lax.axis_index("s") * NC + lax.axis_index("c")
    row0 = wid * ROWS_W

    pltpu.sync_copy(pep_x2.at[pl.ds(row0, ROWS_W)], pep_idx2_v)

    @pl.when(lax.axis_index("s") == 0)
    def _stage_table():
        pltpu.sync_copy(pep_tab, pep_tab_v)

    plsc.subcore_barrier()

    def mask_body(b, carry):
        mv = pep_idx2_v[b, pl.ds(PAD, 16)]
        m = jnp.where(mv != jnp.zeros((16,), jnp.int32),
                      jnp.ones((16,), jnp.int32),
                      jnp.zeros((16,), jnp.int32))
        msk_v[pl.ds(b * CORE_LEN, 16)] = m
        return carry

    lax.fori_loop(0, ROWS_W, mask_body, 0)
    mcopy = pltpu.async_copy(msk_v.at[pl.ds(0, MSK_W)],
                             msk_out.at[pl.ds(wid * MSK_W, MSK_W)], msem)

    def body(gg, carry):
        g0 = gg * NBUF
        gcs = []
        for s in range(NBUF):
            b0 = (g0 + s) * GP
            cps = []
            for r in range(GP):
                cps.append(pltpu.async_copy(
                    pep_tab_v.at[pep_idx2_v.at[b0 + r]], bufs[s].at[r],
                    gsems[s]))
            gcs.append(cps)
        ocs = []
        for s in range(NBUF):
            for cp in gcs[s]:
                cp.wait()
            b0 = (g0 + s) * GP
            ocs.append(pltpu.async_copy(
                bufs[s], pep_out.at[pl.ds(row0 + b0, GP)], osems[s]))
        for oc in ocs:
            oc.wait()
        return carry

    lax.fori_loop(0, PEP_ITERS, body, 0)
    mcopy.wait()


GM = 2
MBUF = 8
MHC_ITERS = ROWS_W // (GM * MBUF)


@functools.partial(
    pl.kernel,
    mesh=_mesh,
    out_type=jax.ShapeDtypeStruct((B, MHC_LEN, EMB), jnp.float32),
    scratch_types=[
        pltpu.VMEM((ROWS_W, MHC_LEN), jnp.int32),
        pltpu.VMEM_SHARED((VOCAB, EMB), jnp.float32),
    ]
    + [pltpu.VMEM((GM, MHC_LEN, EMB), jnp.float32) for _ in range(MBUF)]
    + [pltpu.SemaphoreType.DMA for _ in range(2 * MBUF)],
)
def _mhc_lookup(mhc_x2, mhc_tab, mhc_out,
                mhc_idx2_v, mhc_tab_v, *bufs_and_sems):
    bufs = list(bufs_and_sems[:MBUF])
    gsems = list(bufs_and_sems[MBUF:2 * MBUF])
    osems = list(bufs_and_sems[2 * MBUF:3 * MBUF])

    wid = lax.axis_index("s") * NC + lax.axis_index("c")
    row0 = wid * ROWS_W

    pltpu.sync_copy(mhc_x2.at[pl.ds(row0, ROWS_W)], mhc_idx2_v)

    @pl.when(lax.axis_index("s") == 0)
    def _stage_table():
        pltpu.sync_copy(mhc_tab, mhc_tab_v)

    plsc.subcore_barrier()

    def body(gg, carry):
        g0 = gg * MBUF
        gcs = []
        for s in range(MBUF):
            b0 = (g0 + s) * GM
            cps = []
            for r in range(GM):
                cps.append(pltpu.async_copy(
                    mhc_tab_v.at[mhc_idx2_v.at[b0 + r]], bufs[s].at[r],
                    gsems[s]))
            gcs.append(cps)
        ocs = []
        for s in range(MBUF):
            for cp in gcs[s]:
                cp.wait()
            b0 = (g0 + s) * GM
            ocs.append(pltpu.async_copy(
                bufs[s], mhc_out.at[pl.ds(row0 + b0, GM)], osems[s]))
        for oc in ocs:
            oc.wait()
        return carry

    lax.fori_loop(0, MHC_ITERS, body, 0)


def kernel(peptide_x, peptide_esm_x, mhc_x, peptide_emb, mhc_emb):
    del peptide_esm_x
    pep_out, msk = _pep_lookup(peptide_x.astype(jnp.int32), peptide_emb)
    mhc_out = _mhc_lookup(mhc_x.astype(jnp.int32), mhc_emb)
    masks = msk.reshape(B, CORE_LEN).astype(jnp.bool_)
    return (pep_out, masks, mhc_out)


# speedup vs baseline: 1.0580x; 1.0108x over previous
import functools
import jax
import jax.numpy as jnp
from jax import lax
from jax.experimental import pallas as pl
from jax.experimental.pallas import tpu as pltpu
from jax.experimental.pallas import tpu_sc as plsc

B = 4096
PEP_LEN = 21
MHC_LEN = 34
CORE_LEN = 15
EMB = 128
PAD = 3
VOCAB = 30

NC = 2
NS = 16
NW = NC * NS

ROWS_W = B // NW
MSK_W = ROWS_W * CORE_LEN

GP = 4
NBUF = 8
PEP_ITERS = ROWS_W // (GP * NBUF)

_mesh = plsc.VectorSubcoreMesh(core_axis_name="c", subcore_axis_name="s")


@functools.partial(
    pl.kernel,
    mesh=_mesh,
    out_type=[
        jax.ShapeDtypeStruct((B, PEP_LEN, EMB), jnp.float32),
        jax.ShapeDtypeStruct((B * CORE_LEN,), jnp.int32),
    ],
    scratch_types=[
        pltpu.VMEM((ROWS_W, PEP_LEN), jnp.int32),
        pltpu.VMEM((MSK_W + 16,), jnp.int32),
        pltpu.VMEM_SHARED((VOCAB, EMB), jnp.float32),
    ]
    + [pltpu.VMEM((GP, PEP_LEN, EMB), jnp.float32) for _ in range(NBUF)]
    + [pltpu.SemaphoreType.DMA for _ in range(2 * NBUF + 1)],
)
def _pep_lookup(pep_x2, pep_tab, pep_out, msk_out,
                pep_idx2_v, msk_v, pep_tab_v, *bufs_and_sems):
    bufs = list(bufs_and_sems[:NBUF])
    gsems = list(bufs_and_sems[NBUF:2 * NBUF])
    osems = list(bufs_and_sems[2 * NBUF:3 * NBUF])
    msem = bufs_and_sems[3 * NBUF]

    wid = lax.axis_index("s") * NC + lax.axis_index("c")
    row0 = wid * ROWS_W

    pltpu.sync_copy(pep_x2.at[pl.ds(row0, ROWS_W)], pep_idx2_v)

    @pl.when(lax.axis_index("s") == 0)
    def _stage_table():
        pltpu.sync_copy(pep_tab, pep_tab_v)

    plsc.subcore_barrier()

    def mask_body(b, carry):
        mv = pep_idx2_v[b, pl.ds(PAD, 16)]
        m = jnp.where(mv != jnp.zeros((16,), jnp.int32),
                      jnp.ones((16,), jnp.int32),
                      jnp.zeros((16,), jnp.int32))
        msk_v[pl.ds(b * CORE_LEN, 16)] = m
        return carry

    lax.fori_loop(0, ROWS_W, mask_body, 0)
    mcopy = pltpu.async_copy(msk_v.at[pl.ds(0, MSK_W)],
                             msk_out.at[pl.ds(wid * MSK_W, MSK_W)], msem)

    def body(gg, carry):
        g0 = gg * NBUF
        gcs = []
        for s in range(NBUF):
            b0 = (g0 + s) * GP
            cps = []
            for r in range(GP):
                cps.append(pltpu.async_copy(
                    pep_tab_v.at[pep_idx2_v.at[b0 + r]], bufs[s].at[r],
                    gsems[s]))
            gcs.append(cps)
        ocs = []
        for s in range(NBUF):
            for cp in gcs[s]:
                cp.wait()
            b0 = (g0 + s) * GP
            ocs.append(pltpu.async_copy(
                bufs[s], pep_out.at[pl.ds(row0 + b0, GP)], osems[s]))
        for oc in ocs:
            oc.wait()
        return carry

    lax.fori_loop(0, PEP_ITERS, body, 0)
    mcopy.wait()


GM = 2
MBUF = 8
MHC_ITERS = ROWS_W // (GM * MBUF)


@functools.partial(
    pl.kernel,
    mesh=_mesh,
    out_type=jax.ShapeDtypeStruct((B, MHC_LEN, EMB), jnp.float32),
    scratch_types=[
        pltpu.VMEM((ROWS_W, MHC_LEN), jnp.int32),
        pltpu.VMEM_SHARED((VOCAB, EMB), jnp.float32),
    ]
    + [pltpu.VMEM((GM, MHC_LEN, EMB), jnp.float32) for _ in range(MBUF)]
    + [pltpu.SemaphoreType.DMA for _ in range(2 * MBUF)],
)
def _mhc_lookup(mhc_x2, mhc_tab, mhc_out,
                mhc_idx2_v, mhc_tab_v, *bufs_and_sems):
    bufs = list(bufs_and_sems[:MBUF])
    gsems = list(bufs_and_sems[MBUF:2 * MBUF])
    osems = list(bufs_and_sems[2 * MBUF:3 * MBUF])

    wid = lax.axis_index("s") * NC + lax.axis_index("c")
    row0 = wid * ROWS_W

    pltpu.sync_copy(mhc_x2.at[pl.ds(row0, ROWS_W)], mhc_idx2_v)

    @pl.when(lax.axis_index("s") == 0)
    def _stage_table():
        pltpu.sync_copy(mhc_tab, mhc_tab_v)

    plsc.subcore_barrier()

    def body(gg, carry):
        g0 = gg * MBUF
        gcs = []
        for s in range(MBUF):
            b0 = (g0 + s) * GM
            cps = []
            for r in range(GM):
                cps.append(pltpu.async_copy(
                    mhc_tab_v.at[mhc_idx2_v.at[b0 + r]], bufs[s].at[r],
                    gsems[s]))
            gcs.append(cps)
        ocs = []
        for s in range(MBUF):
            for cp in gcs[s]:
                cp.wait()
            b0 = (g0 + s) * GM
            ocs.append(pltpu.async_copy(
                bufs[s], mhc_out.at[pl.ds(row0 + b0, GM)], osems[s]))
        for oc in ocs:
            oc.wait()
        return carry

    lax.fori_loop(0, MHC_ITERS, body, 0)


def kernel(peptide_x, peptide_esm_x, mhc_x, peptide_emb, mhc_emb):
    del peptide_esm_x
    pep_out, msk = _pep_lookup(peptide_x.astype(jnp.int32), peptide_emb)
    mhc_out = _mhc_lookup(mhc_x.astype(jnp.int32), mhc_emb)
    masks = msk.reshape(B, CORE_LEN).astype(jnp.bool_)
    return (pep_out, masks, mhc_out)


# speedup vs baseline: 1.0605x; 1.0023x over previous
import functools
import jax
import jax.numpy as jnp
from jax import lax
from jax.experimental import pallas as pl
from jax.experimental.pallas import tpu as pltpu
from jax.experimental.pallas import tpu_sc as plsc

B = 4096
PEP_LEN = 21
MHC_LEN = 34
CORE_LEN = 15
EMB = 128
PAD = 3
VOCAB = 30

NC = 2
NS = 16
NW = NC * NS

ROWS_W = B // NW
MSK_W = ROWS_W * CORE_LEN

GP = 4
NBUF = 8
PEP_ITERS = ROWS_W // (GP * NBUF)

_mesh = plsc.VectorSubcoreMesh(core_axis_name="c", subcore_axis_name="s")


@functools.partial(
    pl.kernel,
    mesh=_mesh,
    out_type=[
        jax.ShapeDtypeStruct((B, PEP_LEN, EMB), jnp.float32),
        jax.ShapeDtypeStruct((B * CORE_LEN,), jnp.int32),
    ],
    scratch_types=[
        pltpu.VMEM((ROWS_W, PEP_LEN), jnp.int32),
        pltpu.VMEM((MSK_W + 16,), jnp.int32),
        pltpu.VMEM_SHARED((VOCAB, EMB), jnp.float32),
    ]
    + [pltpu.VMEM((GP, PEP_LEN, EMB), jnp.float32) for _ in range(NBUF)]
    + [pltpu.SemaphoreType.DMA for _ in range(2 * NBUF + 1)],
)
def _pep_lookup(pep_x2, pep_tab, pep_out, msk_out,
                pep_idx2_v, msk_v, pep_tab_v, *bufs_and_sems):
    bufs = list(bufs_and_sems[:NBUF])
    gsems = list(bufs_and_sems[NBUF:2 * NBUF])
    osems = list(bufs_and_sems[2 * NBUF:3 * NBUF])
    msem = bufs_and_sems[3 * NBUF]

    wid = lax.axis_index("s") * NC + lax.axis_index("c")
    row0 = wid * ROWS_W

    pltpu.sync_copy(pep_x2.at[pl.ds(row0, ROWS_W)], pep_idx2_v)

    @pl.when(lax.axis_index("s") == 0)
    def _stage_table():
        pltpu.sync_copy(pep_tab, pep_tab_v)

    plsc.subcore_barrier()

    def mask_body(b, carry):
        mv = pep_idx2_v[b, pl.ds(PAD, 16)]
        m = jnp.where(mv != jnp.zeros((16,), jnp.int32),
                      jnp.ones((16,), jnp.int32),
                      jnp.zeros((16,), jnp.int32))
        msk_v[pl.ds(b * CORE_LEN, 16)] = m
        return carry

    lax.fori_loop(0, ROWS_W, mask_body, 0)
    mcopy = pltpu.async_copy(msk_v.at[pl.ds(0, MSK_W)],
                             msk_out.at[pl.ds(wid * MSK_W, MSK_W)], msem)

    half = NBUF // 2

    def run_half(hs, g0):
        for s in range(hs, hs + half):
            b0 = (g0 + s - hs) * GP
            cps = [pltpu.async_copy(
                pep_tab_v.at[pep_idx2_v.at[b0 + r]], bufs[s].at[r],
                gsems[s]) for r in range(GP)]
            for cp in cps:
                cp.wait()
            pltpu.async_copy(bufs[s], pep_out.at[pl.ds(row0 + b0, GP)],
                             osems[s])

    def drain_half(hs):
        for s in range(hs, hs + half):
            pltpu.make_async_copy(
                bufs[s], pep_out.at[pl.ds(row0, GP)], osems[s]).wait()

    def body(gg, carry):
        @pl.when(gg > 0)
        def _drain_a():
            drain_half(0)

        run_half(0, gg * NBUF)

        @pl.when(gg > 0)
        def _drain_b():
            drain_half(half)

        run_half(half, gg * NBUF + half)
        return carry

    lax.fori_loop(0, PEP_ITERS, body, 0)
    drain_half(0)
    drain_half(half)
    mcopy.wait()


GM = 2
MBUF = 8
MHC_ITERS = ROWS_W // (GM * MBUF)


@functools.partial(
    pl.kernel,
    mesh=_mesh,
    out_type=jax.ShapeDtypeStruct((B, MHC_LEN, EMB), jnp.float32),
    scratch_types=[
        pltpu.VMEM((ROWS_W, MHC_LEN), jnp.int32),
        pltpu.VMEM_SHARED((VOCAB, EMB), jnp.float32),
    ]
    + [pltpu.VMEM((GM, MHC_LEN, EMB), jnp.float32) for _ in range(MBUF)]
    + [pltpu.SemaphoreType.DMA for _ in range(2 * MBUF)],
)
def _mhc_lookup(mhc_x2, mhc_tab, mhc_out,
                mhc_idx2_v, mhc_tab_v, *bufs_and_sems):
    bufs = list(bufs_and_sems[:MBUF])
    gsems = list(bufs_and_sems[MBUF:2 * MBUF])
    osems = list(bufs_and_sems[2 * MBUF:3 * MBUF])

    wid = lax.axis_index("s") * NC + lax.axis_index("c")
    row0 = wid * ROWS_W

    pltpu.sync_copy(mhc_x2.at[pl.ds(row0, ROWS_W)], mhc_idx2_v)

    @pl.when(lax.axis_index("s") == 0)
    def _stage_table():
        pltpu.sync_copy(mhc_tab, mhc_tab_v)

    plsc.subcore_barrier()

    half = MBUF // 2

    def run_half(hs, g0):
        for s in range(hs, hs + half):
            b0 = (g0 + s - hs) * GM
            cps = [pltpu.async_copy(
                mhc_tab_v.at[mhc_idx2_v.at[b0 + r]], bufs[s].at[r],
                gsems[s]) for r in range(GM)]
            for cp in cps:
                cp.wait()
            pltpu.async_copy(bufs[s], mhc_out.at[pl.ds(row0 + b0, GM)],
                             osems[s])

    def drain_half(hs):
        for s in range(hs, hs + half):
            pltpu.make_async_copy(
                bufs[s], mhc_out.at[pl.ds(row0, GM)], osems[s]).wait()

    def body(gg, carry):
        @pl.when(gg > 0)
        def _drain_a():
            drain_half(0)

        run_half(0, gg * MBUF)

        @pl.when(gg > 0)
        def _drain_b():
            drain_half(half)

        run_half(half, gg * MBUF + half)
        return carry

    lax.fori_loop(0, MHC_ITERS, body, 0)
    drain_half(0)
    drain_half(half)


def kernel(peptide_x, peptide_esm_x, mhc_x, peptide_emb, mhc_emb):
    del peptide_esm_x
    pep_out, msk = _pep_lookup(peptide_x.astype(jnp.int32), peptide_emb)
    mhc_out = _mhc_lookup(mhc_x.astype(jnp.int32), mhc_emb)
    masks = msk.reshape(B, CORE_LEN).astype(jnp.bool_)
    return (pep_out, masks, mhc_out)
